# baseline (speedup 1.0000x reference)
import functools

import jax
import jax.numpy as jnp
from jax import lax
from jax.experimental import pallas as pl
from jax.experimental.pallas import tpu as pltpu
from jax.experimental.pallas import tpu_sc as plsc

NC = 2
NS = 16
NW = NC * NS
CHUNK = 128
ZB = 128
F = 16
NBUF = 8


def _tile_ids():
    c = lax.axis_index("c")
    s = lax.axis_index("s")
    return c, s, c * NS + s


def _zero_acc(zeros_hbm, zeros_v, acc_sh, s, stripe, zsem):
    pltpu.sync_copy(zeros_hbm, zeros_v)
    nz = stripe // ZB
    for z in range(nz):
        pltpu.async_copy(zeros_v, acc_sh.at[pl.ds(s * stripe + z * ZB, ZB)],
                         zsem)
    for z in range(nz):
        pltpu.make_async_copy(
            zeros_v, acc_sh.at[pl.ds(s * stripe + z * ZB, ZB)], zsem).wait()


def _write_out(acc_sh, stripe_v, out_hbm, c, s, stripe):
    pltpu.sync_copy(acc_sh.at[pl.ds(s * stripe, stripe)], stripe_v)
    pltpu.sync_copy(stripe_v, out_hbm.at[c, pl.ds(s * stripe, stripe)])


def _sc_deg_body(npad, kch, dst_hbm, ones_hbm, zeros_hbm, out_hbm,
                 idx_v, ones_v, zeros_v, stripe_v, acc_sh, sem, zsem):
    stripe = npad // NS
    c, s, wid = _tile_ids()
    pltpu.sync_copy(dst_hbm.at[wid], idx_v)
    pltpu.sync_copy(ones_hbm, ones_v)
    _zero_acc(zeros_hbm, zeros_v, acc_sh, s, stripe, zsem)
    plsc.subcore_barrier()

    def fire(j, carry):
        pltpu.async_copy(ones_v, acc_sh.at[idx_v.at[j]], sem, add=True)
        return carry

    lax.fori_loop(0, kch, fire, 0)

    def drain(j, carry):
        pltpu.make_async_copy(ones_v, acc_sh.at[idx_v.at[j]], sem).wait()
        return carry

    lax.fori_loop(0, kch, drain, 0)
    plsc.subcore_barrier()
    _write_out(acc_sh, stripe_v, out_hbm, c, s, stripe)


def _sc_agg_body(npad, kch, g_hbm, src_hbm, dst_hbm, zeros_hbm, out_hbm,
                 sidx_v, didx_v, rows_v, zeros_v, stripe_v, acc_sh, *sems):
    gsem = sems[:NBUF]
    ssem = sems[NBUF:2 * NBUF]
    zsem = sems[2 * NBUF]
    stripe = npad // NS
    c, s, wid = _tile_ids()
    pltpu.sync_copy(src_hbm.at[wid], sidx_v)
    pltpu.sync_copy(dst_hbm.at[wid], didx_v)
    for b in range(NBUF):
        pltpu.async_copy(g_hbm.at[sidx_v.at[b]], rows_v.at[b], gsem[b])
    _zero_acc(zeros_hbm, zeros_v, acc_sh, s, stripe, zsem)
    plsc.subcore_barrier()

    no = kch // NBUF

    def outer(g, carry):
        base = g * NBUF
        for b in range(NBUF):
            j = base + b
            pltpu.make_async_copy(
                g_hbm.at[sidx_v.at[j]], rows_v.at[b], gsem[b]).wait()
            pltpu.async_copy(rows_v.at[b], acc_sh.at[didx_v.at[j]], ssem[b],
                             add=True)
        for b in range(NBUF):
            j = base + b
            jn = jnp.where(j + NBUF < kch, j + NBUF, j)
            pltpu.make_async_copy(
                rows_v.at[b], acc_sh.at[didx_v.at[j]], ssem[b]).wait()
            pltpu.async_copy(g_hbm.at[sidx_v.at[jn]], rows_v.at[b], gsem[b])
        return carry

    lax.fori_loop(0, no, outer, 0)
    for b in range(NBUF):
        pltpu.make_async_copy(
            g_hbm.at[sidx_v.at[b]], rows_v.at[b], gsem[b]).wait()
    plsc.subcore_barrier()
    _write_out(acc_sh, stripe_v, out_hbm, c, s, stripe)


_SC_PARAMS = None


def _sc_mesh():
    return plsc.VectorSubcoreMesh(core_axis_name="c", subcore_axis_name="s",
                                  num_cores=NC, num_subcores=NS)


def _sc_deg(dst3, ones_c, zeros_c, npad, kch):
    return pl.kernel(
        functools.partial(_sc_deg_body, npad, kch),
        out_type=jax.ShapeDtypeStruct((NC, npad, F), jnp.float32),
        mesh=_sc_mesh(),
        scratch_types=[
            pltpu.VMEM((kch, CHUNK), jnp.int32),
            pltpu.VMEM((CHUNK, F), jnp.float32),
            pltpu.VMEM((ZB, F), jnp.float32),
            pltpu.VMEM((npad // NS, F), jnp.float32),
            pltpu.VMEM_SHARED((npad, F), jnp.float32),
            pltpu.SemaphoreType.DMA,
            pltpu.SemaphoreType.DMA,
        ],
        compiler_params=pltpu.CompilerParams(use_tc_tiling_on_sc=False),
    )(dst3, ones_c, zeros_c)


def _sc_agg(g, src3, dst3, zeros_c, npad, kch):
    return pl.kernel(
        functools.partial(_sc_agg_body, npad, kch),
        out_type=jax.ShapeDtypeStruct((NC, npad, F), jnp.float32),
        mesh=_sc_mesh(),
        scratch_types=[
            pltpu.VMEM((kch, CHUNK), jnp.int32),
            pltpu.VMEM((kch, CHUNK), jnp.int32),
            pltpu.VMEM((NBUF, CHUNK, F), jnp.float32),
            pltpu.VMEM((ZB, F), jnp.float32),
            pltpu.VMEM((npad // NS, F), jnp.float32),
            pltpu.VMEM_SHARED((npad, F), jnp.float32),
        ] + [pltpu.SemaphoreType.DMA] * (2 * NBUF + 1),
        compiler_params=pltpu.CompilerParams(use_tc_tiling_on_sc=False),
    )(g, src3, dst3, zeros_c)


def _mm1_body(x_ref, w_ref, h_ref):
    h_ref[...] = jnp.dot(x_ref[...], w_ref[...],
                         preferred_element_type=jnp.float32)


def _scale1_body(h_ref, dp_ref, g_ref, s_ref, invs_ref, invd_ref):
    deg = dp_ref[0] + dp_ref[1] + 1.0
    invs = lax.rsqrt(deg)
    invd = 1.0 / deg
    h = h_ref[...]
    g_ref[...] = h * invs
    s_ref[...] = h * invd
    invs_ref[...] = invs
    invd_ref[...] = invd


def _mm2_body(ap_ref, s1_ref, invs_ref, invd_ref, b1_ref, w2_ref,
              g2_ref, s2_ref):
    invs = invs_ref[...]
    agg = invs * (ap_ref[0] + ap_ref[1]) + s1_ref[...] + b1_ref[...]
    z = jnp.maximum(agg, 0.0)
    h2 = jnp.dot(z, w2_ref[...], preferred_element_type=jnp.float32)
    g2_ref[...] = h2 * invs
    s2_ref[...] = h2 * invd_ref[...]


def _out_body(ap_ref, s2_ref, invs_ref, b2_ref, o_ref):
    y = (invs_ref[...] * (ap_ref[0] + ap_ref[1]) + s2_ref[...] + b2_ref[...])
    m = jnp.max(y, axis=1, keepdims=True)
    lse = jnp.log(jnp.sum(jnp.exp(y - m), axis=1, keepdims=True))
    o_ref[...] = y - m - lse


def _row_spec(r):
    return pl.BlockSpec((r, F), lambda i: (i, 0))


def _pair_spec(r):
    return pl.BlockSpec((2, r, F), lambda i: (0, i, 0))


def _tc_mm1(x, W1, n, r):
    dcol = x.shape[1]
    return pl.pallas_call(
        _mm1_body,
        grid=(n // r,),
        in_specs=[
            pl.BlockSpec((r, dcol), lambda i: (i, 0)),
            pl.BlockSpec((dcol, F), lambda i: (0, 0)),
        ],
        out_specs=_row_spec(r),
        out_shape=jax.ShapeDtypeStruct((n, F), jnp.float32),
    )(x, W1)


def _tc_scale1(h, degp, n, r):
    return pl.pallas_call(
        _scale1_body,
        grid=(n // r,),
        in_specs=[_row_spec(r), _pair_spec(r)],
        out_specs=[_row_spec(r)] * 4,
        out_shape=[jax.ShapeDtypeStruct((n, F), jnp.float32)] * 4,
    )(h, degp)


def _tc_mm2(aggp, s1, invs, invd, b1r, W2, n, r):
    return pl.pallas_call(
        _mm2_body,
        grid=(n // r,),
        in_specs=[
            _pair_spec(r), _row_spec(r), _row_spec(r), _row_spec(r),
            pl.BlockSpec((1, F), lambda i: (0, 0)),
            pl.BlockSpec((F, F), lambda i: (0, 0)),
        ],
        out_specs=[_row_spec(r)] * 2,
        out_shape=[jax.ShapeDtypeStruct((n, F), jnp.float32)] * 2,
    )(aggp, s1, invs, invd, b1r, W2)


def _tc_out(aggp, s2, invs, b2r, n, r):
    return pl.pallas_call(
        _out_body,
        grid=(n // r,),
        in_specs=[
            _pair_spec(r), _row_spec(r), _row_spec(r),
            pl.BlockSpec((1, F), lambda i: (0, 0)),
        ],
        out_specs=_row_spec(r),
        out_shape=jax.ShapeDtypeStruct((n, F), jnp.float32),
    )(aggp, s2, invs, b2r)


def kernel(x, edge_index, W1, b1, W2, b2):
    n = x.shape[0]
    e = edge_index.shape[1]
    npad = -(-(n + 1) // (NS * ZB)) * (NS * ZB)
    kch = -(-e // (NW * CHUNK * NBUF)) * NBUF
    epad = NW * kch * CHUNK
    pad = epad - e

    src = edge_index[0]
    dst = edge_index[1]
    dump = n + jnp.arange(pad, dtype=jnp.int32) % (npad - n)
    src3 = jnp.concatenate([src, jnp.zeros((pad,), jnp.int32)]).reshape(
        NW, kch, CHUNK)
    dst3 = jnp.concatenate([dst, dump]).reshape(NW, kch, CHUNK)
    ones_c = jnp.ones((CHUNK, F), jnp.float32)
    zeros_c = jnp.zeros((ZB, F), jnp.float32)

    r = 2000

    degp = _sc_deg(dst3, ones_c, zeros_c, npad, kch)
    h1 = _tc_mm1(x, W1, n, r)
    g1, s1, invs, invd = _tc_scale1(h1, degp, n, r)
    aggp1 = _sc_agg(g1, src3, dst3, zeros_c, npad, kch)
    g2, s2 = _tc_mm2(aggp1, s1, invs, invd, b1.reshape(1, F), W2, n, r)
    aggp2 = _sc_agg(g2, src3, dst3, zeros_c, npad, kch)
    return _tc_out(aggp2, s2, invs, b2.reshape(1, F), n, r)


# speedup vs baseline: 1.3402x; 1.3402x over previous
import functools

import jax
import jax.numpy as jnp
from jax import lax
from jax.experimental import pallas as pl
from jax.experimental.pallas import tpu as pltpu
from jax.experimental.pallas import tpu_sc as plsc

NC = 2
NS = 16
NW = NC * NS
CHUNK = 128
ZB = 128
F = 16
NBUF = 8


def _tile_ids():
    c = lax.axis_index("c")
    s = lax.axis_index("s")
    return c, s, c * NS + s


def _zero_acc(zeros_hbm, zeros_v, acc_sh, s, stripe, zsem):
    pltpu.sync_copy(zeros_hbm, zeros_v)
    nz = stripe // ZB
    for z in range(nz):
        pltpu.async_copy(zeros_v, acc_sh.at[pl.ds(s * stripe + z * ZB, ZB)],
                         zsem)
    for z in range(nz):
        pltpu.make_async_copy(
            zeros_v, acc_sh.at[pl.ds(s * stripe + z * ZB, ZB)], zsem).wait()


def _write_out(acc_sh, stripe_v, out_hbm, c, s, stripe):
    pltpu.sync_copy(acc_sh.at[pl.ds(s * stripe, stripe)], stripe_v)
    pltpu.sync_copy(stripe_v, out_hbm.at[c, pl.ds(s * stripe, stripe)])


def _sc_deg_body(npad, kch, dst_hbm, ones_hbm, zeros_hbm, out_hbm,
                 idx_v, ones_v, zeros_v, stripe_v, acc_sh, sem, zsem):
    stripe = npad // NS
    c, s, wid = _tile_ids()
    pltpu.sync_copy(dst_hbm.at[wid], idx_v)
    pltpu.sync_copy(ones_hbm, ones_v)
    _zero_acc(zeros_hbm, zeros_v, acc_sh, s, stripe, zsem)
    plsc.subcore_barrier()

    def fire(j, carry):
        pltpu.async_copy(ones_v, acc_sh.at[idx_v.at[j]], sem, add=True)
        return carry

    lax.fori_loop(0, kch, fire, 0)

    def drain(j, carry):
        pltpu.make_async_copy(ones_v, acc_sh.at[idx_v.at[j]], sem).wait()
        return carry

    lax.fori_loop(0, kch, drain, 0)
    plsc.subcore_barrier()
    _write_out(acc_sh, stripe_v, out_hbm, c, s, stripe)


def _sc_agg_body(npad, kch, rpt, g_hbm, src_hbm, dst_hbm, zeros_hbm, out_hbm,
                 sidx_v, didx_v, rows_v, zeros_v, stripe_v, acc_sh, g_sh,
                 *sems):
    gsem = sems[:NBUF]
    ssem = sems[NBUF:2 * NBUF]
    zsem = sems[2 * NBUF]
    stripe = npad // NS
    c, s, wid = _tile_ids()
    pltpu.sync_copy(src_hbm.at[wid], sidx_v)
    pltpu.sync_copy(dst_hbm.at[wid], didx_v)
    pltpu.sync_copy(g_hbm.at[pl.ds(s * rpt, rpt)], stripe_v.at[pl.ds(0, rpt)])
    pltpu.sync_copy(stripe_v.at[pl.ds(0, rpt)], g_sh.at[pl.ds(s * rpt, rpt)])
    _zero_acc(zeros_hbm, zeros_v, acc_sh, s, stripe, zsem)
    plsc.subcore_barrier()
    for b in range(NBUF):
        pltpu.async_copy(g_sh.at[sidx_v.at[b]], rows_v.at[b], gsem[b])

    no = kch // NBUF

    def outer(g, carry):
        base = g * NBUF
        for b in range(NBUF):
            j = base + b
            pltpu.make_async_copy(
                g_sh.at[sidx_v.at[j]], rows_v.at[b], gsem[b]).wait()
            pltpu.async_copy(rows_v.at[b], acc_sh.at[didx_v.at[j]], ssem[b],
                             add=True)
        for b in range(NBUF):
            j = base + b
            jn = jnp.where(j + NBUF < kch, j + NBUF, j)
            pltpu.make_async_copy(
                rows_v.at[b], acc_sh.at[didx_v.at[j]], ssem[b]).wait()
            pltpu.async_copy(g_sh.at[sidx_v.at[jn]], rows_v.at[b], gsem[b])
        return carry

    lax.fori_loop(0, no, outer, 0)
    for b in range(NBUF):
        pltpu.make_async_copy(
            g_sh.at[sidx_v.at[b]], rows_v.at[b], gsem[b]).wait()
    plsc.subcore_barrier()
    _write_out(acc_sh, stripe_v, out_hbm, c, s, stripe)


_SC_PARAMS = None


def _sc_mesh():
    return plsc.VectorSubcoreMesh(core_axis_name="c", subcore_axis_name="s",
                                  num_cores=NC, num_subcores=NS)


def _sc_deg(dst3, ones_c, zeros_c, npad, kch):
    return pl.kernel(
        functools.partial(_sc_deg_body, npad, kch),
        out_type=jax.ShapeDtypeStruct((NC, npad, F), jnp.float32),
        mesh=_sc_mesh(),
        scratch_types=[
            pltpu.VMEM((kch, CHUNK), jnp.int32),
            pltpu.VMEM((CHUNK, F), jnp.float32),
            pltpu.VMEM((ZB, F), jnp.float32),
            pltpu.VMEM((npad // NS, F), jnp.float32),
            pltpu.VMEM_SHARED((npad, F), jnp.float32),
            pltpu.SemaphoreType.DMA,
            pltpu.SemaphoreType.DMA,
        ],
        compiler_params=pltpu.CompilerParams(use_tc_tiling_on_sc=False),
    )(dst3, ones_c, zeros_c)


def _sc_agg(g, src3, dst3, zeros_c, npad, kch):
    n = g.shape[0]
    rpt = n // NS
    return pl.kernel(
        functools.partial(_sc_agg_body, npad, kch, rpt),
        out_type=jax.ShapeDtypeStruct((NC, npad, F), jnp.float32),
        mesh=_sc_mesh(),
        scratch_types=[
            pltpu.VMEM((kch, CHUNK), jnp.int32),
            pltpu.VMEM((kch, CHUNK), jnp.int32),
            pltpu.VMEM((NBUF, CHUNK, F), jnp.float32),
            pltpu.VMEM((ZB, F), jnp.float32),
            pltpu.VMEM((npad // NS, F), jnp.float32),
            pltpu.VMEM_SHARED((npad, F), jnp.float32),
            pltpu.VMEM_SHARED((npad, F), jnp.float32),
        ] + [pltpu.SemaphoreType.DMA] * (2 * NBUF + 1),
        compiler_params=pltpu.CompilerParams(use_tc_tiling_on_sc=False),
    )(g, src3, dst3, zeros_c)


def _mm1_body(x_ref, w_ref, h_ref):
    h_ref[...] = jnp.dot(x_ref[...], w_ref[...],
                         preferred_element_type=jnp.float32)


def _scale1_body(h_ref, dp_ref, g_ref, s_ref, invs_ref, invd_ref):
    deg = dp_ref[0] + dp_ref[1] + 1.0
    invs = lax.rsqrt(deg)
    invd = 1.0 / deg
    h = h_ref[...]
    g_ref[...] = h * invs
    s_ref[...] = h * invd
    invs_ref[...] = invs
    invd_ref[...] = invd


def _mm2_body(ap_ref, s1_ref, invs_ref, invd_ref, b1_ref, w2_ref,
              g2_ref, s2_ref):
    invs = invs_ref[...]
    agg = invs * (ap_ref[0] + ap_ref[1]) + s1_ref[...] + b1_ref[...]
    z = jnp.maximum(agg, 0.0)
    h2 = jnp.dot(z, w2_ref[...], preferred_element_type=jnp.float32)
    g2_ref[...] = h2 * invs
    s2_ref[...] = h2 * invd_ref[...]


def _out_body(ap_ref, s2_ref, invs_ref, b2_ref, o_ref):
    y = (invs_ref[...] * (ap_ref[0] + ap_ref[1]) + s2_ref[...] + b2_ref[...])
    m = jnp.max(y, axis=1, keepdims=True)
    lse = jnp.log(jnp.sum(jnp.exp(y - m), axis=1, keepdims=True))
    o_ref[...] = y - m - lse


def _row_spec(r):
    return pl.BlockSpec((r, F), lambda i: (i, 0))


def _pair_spec(r):
    return pl.BlockSpec((2, r, F), lambda i: (0, i, 0))


def _tc_mm1(x, W1, n, r):
    dcol = x.shape[1]
    return pl.pallas_call(
        _mm1_body,
        grid=(n // r,),
        in_specs=[
            pl.BlockSpec((r, dcol), lambda i: (i, 0)),
            pl.BlockSpec((dcol, F), lambda i: (0, 0)),
        ],
        out_specs=_row_spec(r),
        out_shape=jax.ShapeDtypeStruct((n, F), jnp.float32),
    )(x, W1)


def _tc_scale1(h, degp, n, r):
    r = n
    return pl.pallas_call(
        _scale1_body,
        grid=(n // r,),
        in_specs=[_row_spec(r), _pair_spec(r)],
        out_specs=[_row_spec(r)] * 4,
        out_shape=[jax.ShapeDtypeStruct((n, F), jnp.float32)] * 4,
    )(h, degp)


def _tc_mm2(aggp, s1, invs, invd, b1r, W2, n, r):
    r = n
    return pl.pallas_call(
        _mm2_body,
        grid=(n // r,),
        in_specs=[
            _pair_spec(r), _row_spec(r), _row_spec(r), _row_spec(r),
            pl.BlockSpec((1, F), lambda i: (0, 0)),
            pl.BlockSpec((F, F), lambda i: (0, 0)),
        ],
        out_specs=[_row_spec(r)] * 2,
        out_shape=[jax.ShapeDtypeStruct((n, F), jnp.float32)] * 2,
    )(aggp, s1, invs, invd, b1r, W2)


def _tc_out(aggp, s2, invs, b2r, n, r):
    r = n
    return pl.pallas_call(
        _out_body,
        grid=(n // r,),
        in_specs=[
            _pair_spec(r), _row_spec(r), _row_spec(r),
            pl.BlockSpec((1, F), lambda i: (0, 0)),
        ],
        out_specs=_row_spec(r),
        out_shape=jax.ShapeDtypeStruct((n, F), jnp.float32),
    )(aggp, s2, invs, b2r)


def kernel(x, edge_index, W1, b1, W2, b2):
    n = x.shape[0]
    e = edge_index.shape[1]
    npad = -(-(n + 1) // (NS * ZB)) * (NS * ZB)
    kch = -(-e // (NW * CHUNK * NBUF)) * NBUF
    epad = NW * kch * CHUNK
    pad = epad - e

    src = edge_index[0]
    dst = edge_index[1]
    dump = n + jnp.arange(pad, dtype=jnp.int32) % (npad - n)
    src3 = jnp.concatenate([src, jnp.zeros((pad,), jnp.int32)]).reshape(
        NW, kch, CHUNK)
    dst3 = jnp.concatenate([dst, dump]).reshape(NW, kch, CHUNK)
    ones_c = jnp.ones((CHUNK, F), jnp.float32)
    zeros_c = jnp.zeros((ZB, F), jnp.float32)

    r = 2000

    degp = _sc_deg(dst3, ones_c, zeros_c, npad, kch)
    h1 = _tc_mm1(x, W1, n, r)
    g1, s1, invs, invd = _tc_scale1(h1, degp, n, r)
    aggp1 = _sc_agg(g1, src3, dst3, zeros_c, npad, kch)
    g2, s2 = _tc_mm2(aggp1, s1, invs, invd, b1.reshape(1, F), W2, n, r)
    aggp2 = _sc_agg(g2, src3, dst3, zeros_c, npad, kch)
    return _tc_out(aggp2, s2, invs, b2.reshape(1, F), n, r)


# speedup vs baseline: 1.8733x; 1.3977x over previous
import functools

import jax
import jax.numpy as jnp
from jax import lax
from jax.experimental import pallas as pl
from jax.experimental.pallas import tpu as pltpu
from jax.experimental.pallas import tpu_sc as plsc

NC = 2
NS = 16
NW = NC * NS
CHUNK = 128
ZB = 128
F = 16
NBUF = 8


def _tile_ids():
    c = lax.axis_index("c")
    s = lax.axis_index("s")
    return c, s, c * NS + s


def _zero_acc(zeros_hbm, zeros_v, acc_sh, s, stripe, zsem):
    pltpu.sync_copy(zeros_hbm, zeros_v)
    nz = stripe // ZB
    for z in range(nz):
        pltpu.async_copy(zeros_v, acc_sh.at[pl.ds(s * stripe + z * ZB, ZB)],
                         zsem)
    for z in range(nz):
        pltpu.make_async_copy(
            zeros_v, acc_sh.at[pl.ds(s * stripe + z * ZB, ZB)], zsem).wait()


def _write_out(acc_sh, stripe_v, out_hbm, c, s, stripe):
    pltpu.sync_copy(acc_sh.at[pl.ds(s * stripe, stripe)], stripe_v)
    pltpu.sync_copy(stripe_v, out_hbm.at[c, pl.ds(s * stripe, stripe)])


def _sc_deg_body(npad, kch, dst_hbm, ones_hbm, zeros_hbm, out_hbm,
                 idx_v, ones_v, zeros_v, stripe_v, acc_sh, sem, zsem):
    stripe = npad // NS
    c, s, wid = _tile_ids()
    pltpu.sync_copy(dst_hbm.at[wid], idx_v)
    pltpu.sync_copy(ones_hbm, ones_v)
    _zero_acc(zeros_hbm, zeros_v, acc_sh, s, stripe, zsem)
    plsc.subcore_barrier()

    def fire(j, carry):
        pltpu.async_copy(ones_v, acc_sh.at[idx_v.at[j]], sem, add=True)
        return carry

    lax.fori_loop(0, kch, fire, 0)

    def drain(j, carry):
        pltpu.make_async_copy(ones_v, acc_sh.at[idx_v.at[j]], sem).wait()
        return carry

    lax.fori_loop(0, kch, drain, 0)
    plsc.subcore_barrier()
    _write_out(acc_sh, stripe_v, out_hbm, c, s, stripe)


def _sc_agg_body(npad, kch, rpt, g_hbm, src_hbm, dst_hbm, zeros_hbm, out_hbm,
                 sidx_v, didx_v, rows_v, zeros_v, stripe_v, acc_sh, g_sh,
                 *sems):
    gsem = sems[:NBUF]
    ssem = sems[NBUF:2 * NBUF]
    zsem = sems[2 * NBUF]
    stripe = npad // NS
    c, s, wid = _tile_ids()
    pltpu.sync_copy(src_hbm.at[wid], sidx_v)
    pltpu.sync_copy(dst_hbm.at[wid], didx_v)
    pltpu.sync_copy(g_hbm.at[pl.ds(s * rpt, rpt)], stripe_v.at[pl.ds(0, rpt)])
    pltpu.sync_copy(stripe_v.at[pl.ds(0, rpt)], g_sh.at[pl.ds(s * rpt, rpt)])
    _zero_acc(zeros_hbm, zeros_v, acc_sh, s, stripe, zsem)
    plsc.subcore_barrier()
    for b in range(NBUF):
        pltpu.async_copy(g_sh.at[sidx_v.at[b]], rows_v.at[b], gsem[b])

    no = kch // NBUF

    def outer(g, carry):
        base = g * NBUF
        for b in range(NBUF):
            j = base + b
            pltpu.make_async_copy(
                g_sh.at[sidx_v.at[j]], rows_v.at[b], gsem[b]).wait()
            pltpu.async_copy(rows_v.at[b], acc_sh.at[didx_v.at[j]], ssem[b],
                             add=True)
        for b in range(NBUF):
            j = base + b
            jn = jnp.where(j + NBUF < kch, j + NBUF, j)
            pltpu.make_async_copy(
                rows_v.at[b], acc_sh.at[didx_v.at[j]], ssem[b]).wait()
            pltpu.async_copy(g_sh.at[sidx_v.at[jn]], rows_v.at[b], gsem[b])
        return carry

    lax.fori_loop(0, no, outer, 0)
    for b in range(NBUF):
        pltpu.make_async_copy(
            g_sh.at[sidx_v.at[b]], rows_v.at[b], gsem[b]).wait()
    plsc.subcore_barrier()
    _write_out(acc_sh, stripe_v, out_hbm, c, s, stripe)


_SC_PARAMS = None


def _sc_mesh():
    return plsc.VectorSubcoreMesh(core_axis_name="c", subcore_axis_name="s",
                                  num_cores=NC, num_subcores=NS)


def _sc_deg(dst3, ones_c, zeros_c, npad, kch):
    return pl.kernel(
        functools.partial(_sc_deg_body, npad, kch),
        out_type=jax.ShapeDtypeStruct((NC, npad, F), jnp.float32),
        mesh=_sc_mesh(),
        scratch_types=[
            pltpu.VMEM((kch, CHUNK), jnp.int32),
            pltpu.VMEM((CHUNK, F), jnp.float32),
            pltpu.VMEM((ZB, F), jnp.float32),
            pltpu.VMEM((npad // NS, F), jnp.float32),
            pltpu.VMEM_SHARED((npad, F), jnp.float32),
            pltpu.SemaphoreType.DMA,
            pltpu.SemaphoreType.DMA,
        ],
        compiler_params=pltpu.CompilerParams(use_tc_tiling_on_sc=False),
    )(dst3, ones_c, zeros_c)


def _sc_agg(g, src3, dst3, zeros_c, npad, kch):
    n = g.shape[0]
    rpt = n // NS
    return pl.kernel(
        functools.partial(_sc_agg_body, npad, kch, rpt),
        out_type=jax.ShapeDtypeStruct((NC, npad, F), jnp.float32),
        mesh=_sc_mesh(),
        scratch_types=[
            pltpu.VMEM((kch, CHUNK), jnp.int32),
            pltpu.VMEM((kch, CHUNK), jnp.int32),
            pltpu.VMEM((NBUF, CHUNK, F), jnp.float32),
            pltpu.VMEM((ZB, F), jnp.float32),
            pltpu.VMEM((npad // NS, F), jnp.float32),
            pltpu.VMEM_SHARED((npad, F), jnp.float32),
            pltpu.VMEM_SHARED((npad, F), jnp.float32),
        ] + [pltpu.SemaphoreType.DMA] * (2 * NBUF + 1),
        compiler_params=pltpu.CompilerParams(use_tc_tiling_on_sc=False),
    )(g, src3, dst3, zeros_c)


def _scale1_body(h_ref, dp_ref, g_ref, s_ref, invs_ref, invd_ref):
    deg = dp_ref[0] + dp_ref[1] + 1.0
    invs = lax.rsqrt(deg)
    invd = 1.0 / deg
    h = h_ref[...]
    g_ref[...] = h * invs
    s_ref[...] = h * invd
    invs_ref[...] = invs
    invd_ref[...] = invd


def _mm2_body(ap_ref, s1_ref, invs_ref, invd_ref, b1_ref, w2_ref,
              g2_ref, s2_ref):
    invs = invs_ref[...]
    agg = invs * (ap_ref[0] + ap_ref[1]) + s1_ref[...] + b1_ref[...]
    z = jnp.maximum(agg, 0.0)
    h2 = jnp.dot(z, w2_ref[...], preferred_element_type=jnp.float32)
    g2_ref[...] = h2 * invs
    s2_ref[...] = h2 * invd_ref[...]


def _out_body(ap_ref, s2_ref, invs_ref, b2_ref, gsum_ref, o_ref):
    y = (invs_ref[...] * (ap_ref[0] + ap_ref[1]) + s2_ref[...] + b2_ref[...])
    m = jnp.max(y, axis=1, keepdims=True)
    e = jnp.exp(y - m)
    se = jnp.dot(e, gsum_ref[...], preferred_element_type=jnp.float32)
    o_ref[...] = y - m - jnp.log(se)


def _mm1_body(x_ref, w_ref, h_ref):
    h_ref[...] = jnp.dot(x_ref[...], w_ref[...],
                         preferred_element_type=jnp.float32)


def _vspec(vn):
    return pl.BlockSpec((vn, 128), lambda: (0, 0))


def _vpair_spec(vn):
    return pl.BlockSpec((2, vn, 128), lambda: (0, 0, 0))


def _tc_mm1(xv, w1bd, vn, r):
    k = xv.shape[1]
    return pl.pallas_call(
        _mm1_body,
        grid=(vn // r,),
        in_specs=[
            pl.BlockSpec((r, k), lambda i: (i, 0)),
            pl.BlockSpec((k, 128), lambda i: (0, 0)),
        ],
        out_specs=pl.BlockSpec((r, 128), lambda i: (i, 0)),
        out_shape=jax.ShapeDtypeStruct((vn, 128), jnp.float32),
    )(xv, w1bd)


def _tc_scale1(hv, degpv, vn):
    return pl.pallas_call(
        _scale1_body,
        in_specs=[_vspec(vn), _vpair_spec(vn)],
        out_specs=[_vspec(vn)] * 4,
        out_shape=[jax.ShapeDtypeStruct((vn, 128), jnp.float32)] * 4,
    )(hv, degpv)


def _tc_mm2(aggpv, s1v, invsv, invdv, b1t, w2bd, vn):
    return pl.pallas_call(
        _mm2_body,
        in_specs=[
            _vpair_spec(vn), _vspec(vn), _vspec(vn), _vspec(vn),
            pl.BlockSpec((1, 128), lambda: (0, 0)),
            pl.BlockSpec((128, 128), lambda: (0, 0)),
        ],
        out_specs=[_vspec(vn)] * 2,
        out_shape=[jax.ShapeDtypeStruct((vn, 128), jnp.float32)] * 2,
    )(aggpv, s1v, invsv, invdv, b1t, w2bd)


def _tc_out(aggpv, s2v, invsv, b2t, gsum, vn):
    return pl.pallas_call(
        _out_body,
        in_specs=[
            _vpair_spec(vn), _vspec(vn), _vspec(vn),
            pl.BlockSpec((1, 128), lambda: (0, 0)),
            pl.BlockSpec((128, 128), lambda: (0, 0)),
        ],
        out_specs=_vspec(vn),
        out_shape=jax.ShapeDtypeStruct((vn, 128), jnp.float32),
    )(aggpv, s2v, invsv, b2t, gsum)


def kernel(x, edge_index, W1, b1, W2, b2):
    n = x.shape[0]
    e = edge_index.shape[1]
    npad = -(-(n + 1) // (NS * ZB)) * (NS * ZB)
    kch = -(-e // (NW * CHUNK * NBUF)) * NBUF
    epad = NW * kch * CHUNK
    pad = epad - e
    vn = n * F // 128
    vp = npad * F // 128

    src = edge_index[0]
    dst = edge_index[1]
    dump = n + jnp.arange(pad, dtype=jnp.int32) % (npad - n)
    src3 = jnp.concatenate([src, jnp.zeros((pad,), jnp.int32)]).reshape(
        NW, kch, CHUNK)
    dst3 = jnp.concatenate([dst, dump]).reshape(NW, kch, CHUNK)
    ones_c = jnp.ones((CHUNK, F), jnp.float32)
    zeros_c = jnp.zeros((ZB, F), jnp.float32)

    eye8 = jnp.eye(8, dtype=jnp.float32)
    w1bd = jnp.kron(eye8, W1)
    w2bd = jnp.kron(eye8, W2)
    gsum = jnp.kron(eye8, jnp.ones((F, F), jnp.float32))
    b1t = jnp.tile(b1, 8).reshape(1, 128)
    b2t = jnp.tile(b2, 8).reshape(1, 128)
    xv = x.reshape(vn, 8 * x.shape[1])

    degp = _sc_deg(dst3, ones_c, zeros_c, npad, kch)
    hv = _tc_mm1(xv, w1bd, vn, vn)
    g1v, s1v, invsv, invdv = _tc_scale1(
        hv, degp.reshape(2, vp, 128)[:, :vn], vn)
    aggp1 = _sc_agg(g1v.reshape(n, F), src3, dst3, zeros_c, npad, kch)
    g2v, s2v = _tc_mm2(aggp1.reshape(2, vp, 128)[:, :vn], s1v, invsv, invdv,
                       b1t, w2bd, vn)
    aggp2 = _sc_agg(g2v.reshape(n, F), src3, dst3, zeros_c, npad, kch)
    outv = _tc_out(aggp2.reshape(2, vp, 128)[:, :vn], s2v, invsv, b2t, gsum, vn)
    return outv.reshape(n, F)


# speedup vs baseline: 2.0496x; 1.0941x over previous
import functools

import jax
import jax.numpy as jnp
from jax import lax
from jax.experimental import pallas as pl
from jax.experimental.pallas import tpu as pltpu
from jax.experimental.pallas import tpu_sc as plsc

NC = 2
NS = 16
NW = NC * NS
CHUNK = 128
ZB = 128
F = 16
NBUF = 8


def _tile_ids():
    c = lax.axis_index("c")
    s = lax.axis_index("s")
    return c, s, c * NS + s


def _zero_acc(zeros_hbm, zeros_v, acc_sh, s, stripe, zsem):
    pltpu.sync_copy(zeros_hbm, zeros_v)
    nz = stripe // ZB
    for z in range(nz):
        pltpu.async_copy(zeros_v, acc_sh.at[pl.ds(s * stripe + z * ZB, ZB)],
                         zsem)
    for z in range(nz):
        pltpu.make_async_copy(
            zeros_v, acc_sh.at[pl.ds(s * stripe + z * ZB, ZB)], zsem).wait()


def _write_out(acc_sh, stripe_v, out_hbm, c, s, stripe):
    pltpu.sync_copy(acc_sh.at[pl.ds(s * stripe, stripe)], stripe_v)
    pltpu.sync_copy(stripe_v, out_hbm.at[c, pl.ds(s * stripe, stripe)])


def _sc_deg_body(npad, kch, edges_hbm, ones_hbm, zeros_hbm, out_hbm,
                 idx_v, ones_v, zeros_v, stripe_v, acc_sh, sem, zsem):
    stripe = npad // NS
    c, s, wid = _tile_ids()
    pltpu.sync_copy(edges_hbm.at[1, wid], idx_v)
    pltpu.sync_copy(ones_hbm, ones_v)
    _zero_acc(zeros_hbm, zeros_v, acc_sh, s, stripe, zsem)
    plsc.subcore_barrier()

    def fire(j, carry):
        pltpu.async_copy(ones_v, acc_sh.at[idx_v.at[j]], sem, add=True)
        return carry

    lax.fori_loop(0, kch, fire, 0)

    def drain(j, carry):
        pltpu.make_async_copy(ones_v, acc_sh.at[idx_v.at[j]], sem).wait()
        return carry

    lax.fori_loop(0, kch, drain, 0)
    plsc.subcore_barrier()
    _write_out(acc_sh, stripe_v, out_hbm, c, s, stripe)


def _sc_agg_body(npad, kch, rpt, g_hbm, edges_hbm, zeros_hbm, out_hbm,
                 sidx_v, didx_v, rows_v, zeros_v, stripe_v, acc_sh, g_sh,
                 *sems):
    gsem = sems[:NBUF]
    ssem = sems[NBUF:2 * NBUF]
    zsem = sems[2 * NBUF]
    stripe = npad // NS
    c, s, wid = _tile_ids()
    pltpu.sync_copy(edges_hbm.at[0, wid], sidx_v)
    pltpu.sync_copy(edges_hbm.at[1, wid], didx_v)
    pltpu.sync_copy(g_hbm.at[pl.ds(s * rpt, rpt)], stripe_v.at[pl.ds(0, rpt)])
    pltpu.sync_copy(stripe_v.at[pl.ds(0, rpt)], g_sh.at[pl.ds(s * rpt, rpt)])
    _zero_acc(zeros_hbm, zeros_v, acc_sh, s, stripe, zsem)
    plsc.subcore_barrier()
    for b in range(NBUF):
        pltpu.async_copy(g_sh.at[sidx_v.at[b]], rows_v.at[b], gsem[b])

    no = kch // NBUF

    def outer(g, carry):
        base = g * NBUF
        for b in range(NBUF):
            j = base + b
            pltpu.make_async_copy(
                g_sh.at[sidx_v.at[j]], rows_v.at[b], gsem[b]).wait()
            pltpu.async_copy(rows_v.at[b], acc_sh.at[didx_v.at[j]], ssem[b],
                             add=True)
        for b in range(NBUF):
            j = base + b
            jn = jnp.where(j + NBUF < kch, j + NBUF, j)
            pltpu.make_async_copy(
                rows_v.at[b], acc_sh.at[didx_v.at[j]], ssem[b]).wait()
            pltpu.async_copy(g_sh.at[sidx_v.at[jn]], rows_v.at[b], gsem[b])
        return carry

    lax.fori_loop(0, no, outer, 0)
    for b in range(NBUF):
        pltpu.make_async_copy(
            g_sh.at[sidx_v.at[b]], rows_v.at[b], gsem[b]).wait()
    plsc.subcore_barrier()
    _write_out(acc_sh, stripe_v, out_hbm, c, s, stripe)


_SC_PARAMS = None


def _sc_mesh():
    return plsc.VectorSubcoreMesh(core_axis_name="c", subcore_axis_name="s",
                                  num_cores=NC, num_subcores=NS)


def _sc_deg(edges4, ones_c, zeros_c, npad, kch):
    return pl.kernel(
        functools.partial(_sc_deg_body, npad, kch),
        out_type=jax.ShapeDtypeStruct((NC, npad, F), jnp.float32),
        mesh=_sc_mesh(),
        scratch_types=[
            pltpu.VMEM((kch, CHUNK), jnp.int32),
            pltpu.VMEM((CHUNK, F), jnp.float32),
            pltpu.VMEM((ZB, F), jnp.float32),
            pltpu.VMEM((npad // NS, F), jnp.float32),
            pltpu.VMEM_SHARED((npad, F), jnp.float32),
            pltpu.SemaphoreType.DMA,
            pltpu.SemaphoreType.DMA,
        ],
        compiler_params=pltpu.CompilerParams(use_tc_tiling_on_sc=False),
    )(edges4, ones_c, zeros_c)


def _sc_agg(g, edges4, zeros_c, npad, kch):
    n = g.shape[0]
    rpt = n // NS
    return pl.kernel(
        functools.partial(_sc_agg_body, npad, kch, rpt),
        out_type=jax.ShapeDtypeStruct((NC, npad, F), jnp.float32),
        mesh=_sc_mesh(),
        scratch_types=[
            pltpu.VMEM((kch, CHUNK), jnp.int32),
            pltpu.VMEM((kch, CHUNK), jnp.int32),
            pltpu.VMEM((NBUF, CHUNK, F), jnp.float32),
            pltpu.VMEM((ZB, F), jnp.float32),
            pltpu.VMEM((npad // NS, F), jnp.float32),
            pltpu.VMEM_SHARED((npad, F), jnp.float32),
            pltpu.VMEM_SHARED((npad, F), jnp.float32),
        ] + [pltpu.SemaphoreType.DMA] * (2 * NBUF + 1),
        compiler_params=pltpu.CompilerParams(use_tc_tiling_on_sc=False),
    )(g, edges4, zeros_c)


def _scale1_body(h_ref, dp_ref, g_ref, s_ref, invs_ref, invd_ref):
    deg = dp_ref[0] + dp_ref[1] + 1.0
    invs = lax.rsqrt(deg)
    invd = 1.0 / deg
    h = h_ref[...]
    g_ref[...] = h * invs
    s_ref[...] = h * invd
    invs_ref[...] = invs
    invd_ref[...] = invd


def _mm2_body(ap_ref, s1_ref, invs_ref, invd_ref, b1_ref, w2_ref,
              g2_ref, s2_ref):
    invs = invs_ref[...]
    agg = invs * (ap_ref[0] + ap_ref[1]) + s1_ref[...] + b1_ref[...]
    z = jnp.maximum(agg, 0.0)
    h2 = jnp.dot(z, w2_ref[...], preferred_element_type=jnp.float32)
    g2_ref[...] = h2 * invs
    s2_ref[...] = h2 * invd_ref[...]


def _out_body(ap_ref, s2_ref, invs_ref, b2_ref, gsum_ref, o_ref):
    y = (invs_ref[...] * (ap_ref[0] + ap_ref[1]) + s2_ref[...] + b2_ref[...])
    m = jnp.max(y, axis=1, keepdims=True)
    e = jnp.exp(y - m)
    se = jnp.dot(e, gsum_ref[...], preferred_element_type=jnp.float32)
    o_ref[...] = y - m - jnp.log(se)


def _mm1_body(x_ref, w_ref, h_ref):
    xv = x_ref[...].reshape(h_ref.shape[0], 8 * x_ref.shape[1])
    h_ref[...] = jnp.dot(xv, w_ref[...], preferred_element_type=jnp.float32)


def _vspec(vn):
    return pl.BlockSpec((vn, 128), lambda: (0, 0))


def _vpair_spec(vn):
    return pl.BlockSpec((2, vn, 128), lambda: (0, 0, 0))


def _tc_mm1(x, w1bd, vn):
    n, d = x.shape
    return pl.pallas_call(
        _mm1_body,
        in_specs=[
            pl.BlockSpec((n, d), lambda: (0, 0)),
            pl.BlockSpec((8 * d, 128), lambda: (0, 0)),
        ],
        out_specs=_vspec(vn),
        out_shape=jax.ShapeDtypeStruct((vn, 128), jnp.float32),
    )(x, w1bd)


def _tc_scale1(hv, degpv, vn):
    return pl.pallas_call(
        _scale1_body,
        in_specs=[_vspec(vn), _vpair_spec(vn)],
        out_specs=[_vspec(vn)] * 4,
        out_shape=[jax.ShapeDtypeStruct((vn, 128), jnp.float32)] * 4,
    )(hv, degpv)


def _tc_mm2(aggpv, s1v, invsv, invdv, b1t, w2bd, vn):
    return pl.pallas_call(
        _mm2_body,
        in_specs=[
            _vpair_spec(vn), _vspec(vn), _vspec(vn), _vspec(vn),
            pl.BlockSpec((1, 128), lambda: (0, 0)),
            pl.BlockSpec((128, 128), lambda: (0, 0)),
        ],
        out_specs=[_vspec(vn)] * 2,
        out_shape=[jax.ShapeDtypeStruct((vn, 128), jnp.float32)] * 2,
    )(aggpv, s1v, invsv, invdv, b1t, w2bd)


def _tc_out(aggpv, s2v, invsv, b2t, gsum, vn):
    return pl.pallas_call(
        _out_body,
        in_specs=[
            _vpair_spec(vn), _vspec(vn), _vspec(vn),
            pl.BlockSpec((1, 128), lambda: (0, 0)),
            pl.BlockSpec((128, 128), lambda: (0, 0)),
        ],
        out_specs=_vspec(vn),
        out_shape=jax.ShapeDtypeStruct((vn, 128), jnp.float32),
    )(aggpv, s2v, invsv, b2t, gsum)


def kernel(x, edge_index, W1, b1, W2, b2):
    n = x.shape[0]
    e = edge_index.shape[1]
    npad = -(-(n + 1) // (NS * ZB)) * (NS * ZB)
    kch = -(-e // (NW * CHUNK * NBUF)) * NBUF
    epad = NW * kch * CHUNK
    pad = epad - e
    vn = n * F // 128
    vp = npad * F // 128

    edges4 = jnp.pad(edge_index, ((0, 0), (0, pad)),
                     constant_values=n).reshape(2, NW, kch, CHUNK)
    ones_c = jnp.ones((CHUNK, F), jnp.float32)
    zeros_c = jnp.zeros((ZB, F), jnp.float32)

    eye8 = jnp.eye(8, dtype=jnp.float32)
    w1bd = jnp.kron(eye8, W1)
    w2bd = jnp.kron(eye8, W2)
    gsum = jnp.kron(eye8, jnp.ones((F, F), jnp.float32))
    b1t = jnp.tile(b1, 8).reshape(1, 128)
    b2t = jnp.tile(b2, 8).reshape(1, 128)
    degp = _sc_deg(edges4, ones_c, zeros_c, npad, kch)
    hv = _tc_mm1(x, w1bd, vn)
    g1v, s1v, invsv, invdv = _tc_scale1(
        hv, degp.reshape(2, vp, 128)[:, :vn], vn)
    aggp1 = _sc_agg(g1v.reshape(n, F), edges4, zeros_c, npad, kch)
    g2v, s2v = _tc_mm2(aggp1.reshape(2, vp, 128)[:, :vn], s1v, invsv, invdv,
                       b1t, w2bd, vn)
    aggp2 = _sc_agg(g2v.reshape(n, F), edges4, zeros_c, npad, kch)
    outv = _tc_out(aggp2.reshape(2, vp, 128)[:, :vn], s2v, invsv, b2t, gsum, vn)
    return outv.reshape(n, F)


# speedup vs baseline: 2.0621x; 1.0061x over previous
import functools

import jax
import jax.numpy as jnp
from jax import lax
from jax.experimental import pallas as pl
from jax.experimental.pallas import tpu as pltpu
from jax.experimental.pallas import tpu_sc as plsc

NC = 2
NS = 16
NW = NC * NS
CHUNK = 128
ZB = 128
F = 16
NBUF = 10


def _tile_ids():
    c = lax.axis_index("c")
    s = lax.axis_index("s")
    return c, s, c * NS + s


def _zero_acc(zeros_hbm, zeros_v, acc_sh, s, stripe, zsem):
    pltpu.sync_copy(zeros_hbm, zeros_v)
    nz = stripe // ZB
    for z in range(nz):
        pltpu.async_copy(zeros_v, acc_sh.at[pl.ds(s * stripe + z * ZB, ZB)],
                         zsem)
    for z in range(nz):
        pltpu.make_async_copy(
            zeros_v, acc_sh.at[pl.ds(s * stripe + z * ZB, ZB)], zsem).wait()


def _write_out(acc_sh, stripe_v, out_hbm, c, s, rpt):
    pltpu.sync_copy(acc_sh.at[pl.ds(s * rpt, rpt)], stripe_v.at[pl.ds(0, rpt)])
    pltpu.sync_copy(stripe_v.at[pl.ds(0, rpt)], out_hbm.at[c, pl.ds(s * rpt, rpt)])


def _sc_deg_body(npad, kch, rpt, edges_hbm, ones_hbm, zeros_hbm, out_hbm,
                 idx_v, ones_v, zeros_v, stripe_v, acc_sh, sem, zsem):
    stripe = npad // NS
    c, s, wid = _tile_ids()
    pltpu.sync_copy(edges_hbm.at[1, wid], idx_v)
    pltpu.sync_copy(ones_hbm, ones_v)
    _zero_acc(zeros_hbm, zeros_v, acc_sh, s, stripe, zsem)
    plsc.subcore_barrier()

    def fire(j, carry):
        pltpu.async_copy(ones_v, acc_sh.at[idx_v.at[j]], sem, add=True)
        return carry

    lax.fori_loop(0, kch, fire, 0)

    def drain(j, carry):
        pltpu.make_async_copy(ones_v, acc_sh.at[idx_v.at[j]], sem).wait()
        return carry

    lax.fori_loop(0, kch, drain, 0)
    plsc.subcore_barrier()
    _write_out(acc_sh, stripe_v, out_hbm, c, s, rpt)


def _sc_agg_body(npad, kch, rpt, g_hbm, edges_hbm, zeros_hbm, out_hbm,
                 sidx_v, didx_v, rows_v, zeros_v, stripe_v, acc_sh, g_sh,
                 *sems):
    gsem = sems[:NBUF]
    ssem = sems[NBUF:2 * NBUF]
    zsem = sems[2 * NBUF]
    stripe = npad // NS
    c, s, wid = _tile_ids()
    pltpu.sync_copy(edges_hbm.at[0, wid], sidx_v)
    pltpu.sync_copy(edges_hbm.at[1, wid], didx_v)
    pltpu.sync_copy(g_hbm.at[pl.ds(s * rpt, rpt)], stripe_v.at[pl.ds(0, rpt)])
    pltpu.sync_copy(stripe_v.at[pl.ds(0, rpt)], g_sh.at[pl.ds(s * rpt, rpt)])
    _zero_acc(zeros_hbm, zeros_v, acc_sh, s, stripe, zsem)
    plsc.subcore_barrier()
    for b in range(NBUF):
        pltpu.async_copy(g_sh.at[sidx_v.at[b]], rows_v.at[b], gsem[b])

    no = kch // NBUF

    def outer(g, carry):
        base = g * NBUF
        for b in range(NBUF):
            j = base + b
            pltpu.make_async_copy(
                g_sh.at[sidx_v.at[j]], rows_v.at[b], gsem[b]).wait()
            pltpu.async_copy(rows_v.at[b], acc_sh.at[didx_v.at[j]], ssem[b],
                             add=True)
        for b in range(NBUF):
            j = base + b
            jn = jnp.where(j + NBUF < kch, j + NBUF, j)
            pltpu.make_async_copy(
                rows_v.at[b], acc_sh.at[didx_v.at[j]], ssem[b]).wait()
            pltpu.async_copy(g_sh.at[sidx_v.at[jn]], rows_v.at[b], gsem[b])
        return carry

    lax.fori_loop(0, no, outer, 0)
    for b in range(NBUF):
        pltpu.make_async_copy(
            g_sh.at[sidx_v.at[b]], rows_v.at[b], gsem[b]).wait()
    plsc.subcore_barrier()
    _write_out(acc_sh, stripe_v, out_hbm, c, s, rpt)


_SC_PARAMS = None


def _sc_mesh():
    return plsc.VectorSubcoreMesh(core_axis_name="c", subcore_axis_name="s",
                                  num_cores=NC, num_subcores=NS)


def _sc_deg(edges4, ones_c, zeros_c, npad, kch, n):
    rpt = n // NS
    return pl.kernel(
        functools.partial(_sc_deg_body, npad, kch, rpt),
        out_type=jax.ShapeDtypeStruct((NC, n, F), jnp.float32),
        mesh=_sc_mesh(),
        scratch_types=[
            pltpu.VMEM((kch, CHUNK), jnp.int32),
            pltpu.VMEM((CHUNK, F), jnp.float32),
            pltpu.VMEM((ZB, F), jnp.float32),
            pltpu.VMEM((npad // NS, F), jnp.float32),
            pltpu.VMEM_SHARED((npad, F), jnp.float32),
            pltpu.SemaphoreType.DMA,
            pltpu.SemaphoreType.DMA,
        ],
        compiler_params=pltpu.CompilerParams(use_tc_tiling_on_sc=False),
    )(edges4, ones_c, zeros_c)


def _sc_agg(g, edges4, zeros_c, npad, kch):
    n = g.shape[0]
    rpt = n // NS
    return pl.kernel(
        functools.partial(_sc_agg_body, npad, kch, rpt),
        out_type=jax.ShapeDtypeStruct((NC, n, F), jnp.float32),
        mesh=_sc_mesh(),
        scratch_types=[
            pltpu.VMEM((kch, CHUNK), jnp.int32),
            pltpu.VMEM((kch, CHUNK), jnp.int32),
            pltpu.VMEM((NBUF, CHUNK, F), jnp.float32),
            pltpu.VMEM((ZB, F), jnp.float32),
            pltpu.VMEM((npad // NS, F), jnp.float32),
            pltpu.VMEM_SHARED((npad, F), jnp.float32),
            pltpu.VMEM_SHARED((npad, F), jnp.float32),
        ] + [pltpu.SemaphoreType.DMA] * (2 * NBUF + 1),
        compiler_params=pltpu.CompilerParams(use_tc_tiling_on_sc=False),
    )(g, edges4, zeros_c)


def _scale1_body(h_ref, dp_ref, g_ref, s_ref, invs_ref, invd_ref):
    deg = dp_ref[0] + dp_ref[1] + 1.0
    invs = lax.rsqrt(deg)
    invd = 1.0 / deg
    h = h_ref[...]
    g_ref[...] = h * invs
    s_ref[...] = h * invd
    invs_ref[...] = invs
    invd_ref[...] = invd


def _mm2_body(ap_ref, s1_ref, invs_ref, invd_ref, b1_ref, w2_ref,
              g2_ref, s2_ref):
    invs = invs_ref[...]
    agg = invs * (ap_ref[0] + ap_ref[1]) + s1_ref[...] + b1_ref[...]
    z = jnp.maximum(agg, 0.0)
    h2 = jnp.dot(z, w2_ref[...], preferred_element_type=jnp.float32)
    g2_ref[...] = h2 * invs
    s2_ref[...] = h2 * invd_ref[...]


def _out_body(ap_ref, s2_ref, invs_ref, b2_ref, gsum_ref, o_ref):
    y = (invs_ref[...] * (ap_ref[0] + ap_ref[1]) + s2_ref[...] + b2_ref[...])
    m = jnp.max(y, axis=1, keepdims=True)
    e = jnp.exp(y - m)
    se = jnp.dot(e, gsum_ref[...], preferred_element_type=jnp.float32)
    o_ref[...] = y - m - jnp.log(se)


def _mm1_body(x_ref, w_ref, h_ref):
    xv = x_ref[...].reshape(h_ref.shape[0], 8 * x_ref.shape[1])
    h_ref[...] = jnp.dot(xv, w_ref[...], preferred_element_type=jnp.float32)


def _vspec(vn):
    return pl.BlockSpec((vn, 128), lambda: (0, 0))


def _vpair_spec(vn):
    return pl.BlockSpec((2, vn, 128), lambda: (0, 0, 0))


def _tc_mm1(x, w1bd, vn):
    n, d = x.shape
    return pl.pallas_call(
        _mm1_body,
        in_specs=[
            pl.BlockSpec((n, d), lambda: (0, 0)),
            pl.BlockSpec((8 * d, 128), lambda: (0, 0)),
        ],
        out_specs=_vspec(vn),
        out_shape=jax.ShapeDtypeStruct((vn, 128), jnp.float32),
    )(x, w1bd)


def _tc_scale1(hv, degpv, vn):
    return pl.pallas_call(
        _scale1_body,
        in_specs=[_vspec(vn), _vpair_spec(vn)],
        out_specs=[_vspec(vn)] * 4,
        out_shape=[jax.ShapeDtypeStruct((vn, 128), jnp.float32)] * 4,
    )(hv, degpv)


def _tc_mm2(aggpv, s1v, invsv, invdv, b1t, w2bd, vn):
    return pl.pallas_call(
        _mm2_body,
        in_specs=[
            _vpair_spec(vn), _vspec(vn), _vspec(vn), _vspec(vn),
            pl.BlockSpec((1, 128), lambda: (0, 0)),
            pl.BlockSpec((128, 128), lambda: (0, 0)),
        ],
        out_specs=[_vspec(vn)] * 2,
        out_shape=[jax.ShapeDtypeStruct((vn, 128), jnp.float32)] * 2,
    )(aggpv, s1v, invsv, invdv, b1t, w2bd)


def _tc_out(aggpv, s2v, invsv, b2t, gsum, vn):
    return pl.pallas_call(
        _out_body,
        in_specs=[
            _vpair_spec(vn), _vspec(vn), _vspec(vn),
            pl.BlockSpec((1, 128), lambda: (0, 0)),
            pl.BlockSpec((128, 128), lambda: (0, 0)),
        ],
        out_specs=_vspec(vn),
        out_shape=jax.ShapeDtypeStruct((vn, 128), jnp.float32),
    )(aggpv, s2v, invsv, b2t, gsum)


def kernel(x, edge_index, W1, b1, W2, b2):
    n = x.shape[0]
    e = edge_index.shape[1]
    npad = -(-(n + 1) // (NS * ZB)) * (NS * ZB)
    kch = -(-e // (NW * CHUNK * NBUF)) * NBUF
    epad = NW * kch * CHUNK
    pad = epad - e
    vn = n * F // 128
    vp = npad * F // 128

    edges4 = jnp.pad(edge_index, ((0, 0), (0, pad)),
                     constant_values=n).reshape(2, NW, kch, CHUNK)
    ones_c = jnp.ones((CHUNK, F), jnp.float32)
    zeros_c = jnp.zeros((ZB, F), jnp.float32)

    eye8 = jnp.eye(8, dtype=jnp.float32)
    w1bd = jnp.kron(eye8, W1)
    w2bd = jnp.kron(eye8, W2)
    gsum = jnp.kron(eye8, jnp.ones((F, F), jnp.float32))
    b1t = jnp.tile(b1, 8).reshape(1, 128)
    b2t = jnp.tile(b2, 8).reshape(1, 128)
    degp = _sc_deg(edges4, ones_c, zeros_c, npad, kch, n)
    hv = _tc_mm1(x, w1bd, vn)
    g1v, s1v, invsv, invdv = _tc_scale1(hv, degp.reshape(2, vn, 128), vn)
    aggp1 = _sc_agg(g1v.reshape(n, F), edges4, zeros_c, npad, kch)
    g2v, s2v = _tc_mm2(aggp1.reshape(2, vn, 128), s1v, invsv, invdv,
                       b1t, w2bd, vn)
    aggp2 = _sc_agg(g2v.reshape(n, F), edges4, zeros_c, npad, kch)
    outv = _tc_out(aggp2.reshape(2, vn, 128), s2v, invsv, b2t, gsum, vn)
    return outv.reshape(n, F)


# speedup vs baseline: 2.0634x; 1.0006x over previous
import functools

import jax
import jax.numpy as jnp
from jax import lax
from jax.experimental import pallas as pl
from jax.experimental.pallas import tpu as pltpu
from jax.experimental.pallas import tpu_sc as plsc

NC = 2
NS = 16
NW = NC * NS
CHUNK = 128
ZB = 128
F = 16
NBUF = 10


def _tile_ids():
    c = lax.axis_index("c")
    s = lax.axis_index("s")
    return c, s, c * NS + s


def _zero_acc(zeros_hbm, zeros_v, acc_sh, s, stripe, zsem):
    pltpu.sync_copy(zeros_hbm, zeros_v)
    nz = stripe // ZB
    for z in range(nz):
        pltpu.async_copy(zeros_v, acc_sh.at[pl.ds(s * stripe + z * ZB, ZB)],
                         zsem)
    for z in range(nz):
        pltpu.make_async_copy(
            zeros_v, acc_sh.at[pl.ds(s * stripe + z * ZB, ZB)], zsem).wait()


def _write_out(acc_sh, stripe_v, out_hbm, c, s, rpt):
    pltpu.sync_copy(acc_sh.at[pl.ds(s * rpt, rpt)], stripe_v.at[pl.ds(0, rpt)])
    pltpu.sync_copy(stripe_v.at[pl.ds(0, rpt)], out_hbm.at[c, pl.ds(s * rpt, rpt)])


def _sc_deg_body(npad, kch, rpt, edges_hbm, ones_hbm, zeros_hbm, out_hbm,
                 idx_v, ones_v, zeros_v, stripe_v, acc_sh, sem, zsem):
    stripe = npad // NS
    c, s, wid = _tile_ids()
    pltpu.sync_copy(edges_hbm.at[1, wid], idx_v)
    pltpu.sync_copy(ones_hbm, ones_v)
    _zero_acc(zeros_hbm, zeros_v, acc_sh, s, stripe, zsem)
    plsc.subcore_barrier()

    def fire(j, carry):
        pltpu.async_copy(ones_v, acc_sh.at[idx_v.at[j]], sem, add=True)
        return carry

    lax.fori_loop(0, kch, fire, 0)

    def drain(j, carry):
        pltpu.make_async_copy(ones_v, acc_sh.at[idx_v.at[j]], sem).wait()
        return carry

    lax.fori_loop(0, kch, drain, 0)
    plsc.subcore_barrier()
    _write_out(acc_sh, stripe_v, out_hbm, c, s, rpt)


def _sc_agg_body(npad, kch, rpt, g_hbm, edges_hbm, zeros_hbm, out_hbm,
                 sidx_v, didx_v, rows_v, zeros_v, stripe_v, acc_sh, g_sh,
                 *sems):
    gsem = sems[:NBUF]
    ssem = sems[NBUF:2 * NBUF]
    zsem = sems[2 * NBUF]
    stripe = npad // NS
    c, s, wid = _tile_ids()
    pltpu.sync_copy(edges_hbm.at[0, wid], sidx_v)
    pltpu.sync_copy(edges_hbm.at[1, wid], didx_v)
    pltpu.sync_copy(g_hbm.at[pl.ds(s * rpt, rpt)], stripe_v.at[pl.ds(0, rpt)])
    pltpu.sync_copy(stripe_v.at[pl.ds(0, rpt)], g_sh.at[pl.ds(s * rpt, rpt)])
    _zero_acc(zeros_hbm, zeros_v, acc_sh, s, stripe, zsem)
    plsc.subcore_barrier()
    for b in range(NBUF):
        pltpu.async_copy(g_sh.at[sidx_v.at[b]], rows_v.at[b], gsem[b])

    no = kch // NBUF

    def outer(g, carry):
        base = g * NBUF
        for b in range(NBUF):
            j = base + b
            pltpu.make_async_copy(
                g_sh.at[sidx_v.at[j]], rows_v.at[b], gsem[b]).wait()
            pltpu.async_copy(rows_v.at[b], acc_sh.at[didx_v.at[j]], ssem[b],
                             add=True)
        for b in range(NBUF):
            j = base + b
            jn = jnp.where(j + NBUF < kch, j + NBUF, j)
            pltpu.make_async_copy(
                rows_v.at[b], acc_sh.at[didx_v.at[j]], ssem[b]).wait()
            pltpu.async_copy(g_sh.at[sidx_v.at[jn]], rows_v.at[b], gsem[b])
        return carry

    lax.fori_loop(0, no, outer, 0)
    for b in range(NBUF):
        pltpu.make_async_copy(
            g_sh.at[sidx_v.at[b]], rows_v.at[b], gsem[b]).wait()
    plsc.subcore_barrier()
    _write_out(acc_sh, stripe_v, out_hbm, c, s, rpt)


def _sc_mesh():
    return plsc.VectorSubcoreMesh(core_axis_name="c", subcore_axis_name="s",
                                  num_cores=NC, num_subcores=NS)


def _sc_deg(edges4, ones_c, zeros_c, npad, kch, n):
    rpt = n // NS
    return pl.kernel(
        functools.partial(_sc_deg_body, npad, kch, rpt),
        out_type=jax.ShapeDtypeStruct((NC, n, F), jnp.float32),
        mesh=_sc_mesh(),
        scratch_types=[
            pltpu.VMEM((kch, CHUNK), jnp.int32),
            pltpu.VMEM((CHUNK, F), jnp.float32),
            pltpu.VMEM((ZB, F), jnp.float32),
            pltpu.VMEM((npad // NS, F), jnp.float32),
            pltpu.VMEM_SHARED((npad, F), jnp.float32),
            pltpu.SemaphoreType.DMA,
            pltpu.SemaphoreType.DMA,
        ],
        compiler_params=pltpu.CompilerParams(use_tc_tiling_on_sc=False),
    )(edges4, ones_c, zeros_c)


def _sc_agg(g, edges4, zeros_c, npad, kch):
    n = g.shape[0]
    rpt = n // NS
    return pl.kernel(
        functools.partial(_sc_agg_body, npad, kch, rpt),
        out_type=jax.ShapeDtypeStruct((NC, n, F), jnp.float32),
        mesh=_sc_mesh(),
        scratch_types=[
            pltpu.VMEM((kch, CHUNK), jnp.int32),
            pltpu.VMEM((kch, CHUNK), jnp.int32),
            pltpu.VMEM((NBUF, CHUNK, F), jnp.float32),
            pltpu.VMEM((ZB, F), jnp.float32),
            pltpu.VMEM((npad // NS, F), jnp.float32),
            pltpu.VMEM_SHARED((npad, F), jnp.float32),
            pltpu.VMEM_SHARED((npad, F), jnp.float32),
        ] + [pltpu.SemaphoreType.DMA] * (2 * NBUF + 1),
        compiler_params=pltpu.CompilerParams(use_tc_tiling_on_sc=False),
    )(g, edges4, zeros_c)


def _scale1_body(h_ref, dp_ref, g_ref, s_ref, invs_ref, invd_ref):
    deg = dp_ref[0] + dp_ref[1] + 1.0
    invs = lax.rsqrt(deg)
    invd = 1.0 / deg
    h = h_ref[...]
    g_ref[...] = h * invs
    s_ref[...] = h * invd
    invs_ref[...] = invs
    invd_ref[...] = invd


def _mm2_body(ap_ref, s1_ref, invs_ref, invd_ref, b1_ref, w2_ref,
              g2_ref, s2_ref):
    invs = invs_ref[...]
    agg = invs * (ap_ref[0] + ap_ref[1]) + s1_ref[...] + b1_ref[...]
    z = jnp.maximum(agg, 0.0)
    h2 = jnp.dot(z, w2_ref[...], preferred_element_type=jnp.float32)
    g2_ref[...] = h2 * invs
    s2_ref[...] = h2 * invd_ref[...]


def _out_body(ap_ref, s2_ref, invs_ref, b2_ref, gsum_ref, o_ref):
    y = (invs_ref[...] * (ap_ref[0] + ap_ref[1]) + s2_ref[...] + b2_ref[...])
    m = jnp.max(y, axis=1, keepdims=True)
    e = jnp.exp(y - m)
    se = jnp.dot(e, gsum_ref[...], preferred_element_type=jnp.float32)
    o_ref[...] = y - m - jnp.log(se)


def _mm1_body(x_ref, w_ref, h_ref):
    xv = x_ref[...].reshape(h_ref.shape[0], 8 * x_ref.shape[1])
    h_ref[...] = jnp.dot(xv, w_ref[...], preferred_element_type=jnp.float32)


def _vspec(vn):
    return pl.BlockSpec((vn, 128), lambda: (0, 0))


def _vpair_spec(vn):
    return pl.BlockSpec((2, vn, 128), lambda: (0, 0, 0))


def _tc_mm1(x, w1bd, vn):
    n, d = x.shape
    return pl.pallas_call(
        _mm1_body,
        in_specs=[
            pl.BlockSpec((n, d), lambda: (0, 0)),
            pl.BlockSpec((8 * d, 128), lambda: (0, 0)),
        ],
        out_specs=_vspec(vn),
        out_shape=jax.ShapeDtypeStruct((vn, 128), jnp.float32),
    )(x, w1bd)


def _tc_scale1(hv, degpv, vn):
    return pl.pallas_call(
        _scale1_body,
        in_specs=[_vspec(vn), _vpair_spec(vn)],
        out_specs=[_vspec(vn)] * 4,
        out_shape=[jax.ShapeDtypeStruct((vn, 128), jnp.float32)] * 4,
    )(hv, degpv)


def _tc_mm2(aggpv, s1v, invsv, invdv, b1t, w2bd, vn):
    return pl.pallas_call(
        _mm2_body,
        in_specs=[
            _vpair_spec(vn), _vspec(vn), _vspec(vn), _vspec(vn),
            pl.BlockSpec((1, 128), lambda: (0, 0)),
            pl.BlockSpec((128, 128), lambda: (0, 0)),
        ],
        out_specs=[_vspec(vn)] * 2,
        out_shape=[jax.ShapeDtypeStruct((vn, 128), jnp.float32)] * 2,
    )(aggpv, s1v, invsv, invdv, b1t, w2bd)


def _tc_out(aggpv, s2v, invsv, b2t, gsum, vn):
    return pl.pallas_call(
        _out_body,
        in_specs=[
            _vpair_spec(vn), _vspec(vn), _vspec(vn),
            pl.BlockSpec((1, 128), lambda: (0, 0)),
            pl.BlockSpec((128, 128), lambda: (0, 0)),
        ],
        out_specs=_vspec(vn),
        out_shape=jax.ShapeDtypeStruct((vn, 128), jnp.float32),
    )(aggpv, s2v, invsv, b2t, gsum)


def kernel(x, edge_index, W1, b1, W2, b2):
    n = x.shape[0]
    e = edge_index.shape[1]
    npad = -(-(n + 1) // (NS * ZB)) * (NS * ZB)
    kch = -(-e // (NW * CHUNK * NBUF)) * NBUF
    epad = NW * kch * CHUNK
    pad = epad - e
    vn = n * F // 128

    edges4 = jnp.pad(edge_index, ((0, 0), (0, pad)),
                     constant_values=n).reshape(2, NW, kch, CHUNK)
    ones_c = jnp.ones((CHUNK, F), jnp.float32)
    zeros_c = jnp.zeros((ZB, F), jnp.float32)

    eye8 = jnp.eye(8, dtype=jnp.float32)
    w1bd = jnp.kron(eye8, W1)
    w2bd = jnp.kron(eye8, W2)
    gsum = jnp.kron(eye8, jnp.ones((F, F), jnp.float32))
    b1t = jnp.tile(b1, 8).reshape(1, 128)
    b2t = jnp.tile(b2, 8).reshape(1, 128)
    degp = _sc_deg(edges4, ones_c, zeros_c, npad, kch, n)
    hv = _tc_mm1(x, w1bd, vn)
    g1v, s1v, invsv, invdv = _tc_scale1(hv, degp.reshape(2, vn, 128), vn)
    aggp1 = _sc_agg(g1v.reshape(n, F), edges4, zeros_c, npad, kch)
    g2v, s2v = _tc_mm2(aggp1.reshape(2, vn, 128), s1v, invsv, invdv,
                       b1t, w2bd, vn)
    aggp2 = _sc_agg(g2v.reshape(n, F), edges4, zeros_c, npad, kch)
    outv = _tc_out(aggp2.reshape(2, vn, 128), s2v, invsv, b2t, gsum, vn)
    return outv.reshape(n, F)
